# Initial kernel scaffold; baseline (speedup 1.0000x reference)
import jax
import jax.numpy as jnp
from jax.experimental import pallas as pl


def kernel(feat, feat_a, adj, adj_feat, graph_neigh, weight_spatial, weight_feat, weight_back, spatial_weight, feature_weight, w_omega, u_omega, fc_w, fc_b, disc_w, disc_b):
    raise NotImplementedError("write your pallas kernel here")


# speedup vs baseline: 2.1984x; 2.1984x over previous
import jax
import jax.numpy as jnp
from jax.experimental import pallas as pl
from jax.experimental.pallas import tpu as pltpu

N = 4096
IN = 512
OUT = 256
HEADS = 4
HD = OUT // HEADS

_BF = jnp.bfloat16
_F32 = jnp.float32

_BM1 = 1024
_BM2, _BK2 = 256, 2048
_BM3, _BK3 = 256, 2048


def _dot(a, b):
    return jnp.dot(a, b, preferred_element_type=_F32)


def _proj_body(feat_ref, feata_ref, wsf_ref, s_ref, f_ref):
    w = wsf_ref[...]
    p = _dot(feat_ref[...], w)
    pa = _dot(feata_ref[...], w)
    s_ref[...] = jnp.concatenate([p[:, :OUT], pa[:, :OUT]], axis=1).astype(_BF)
    f_ref[...] = jnp.concatenate([p[:, OUT:], pa[:, OUT:]], axis=1).astype(_BF)


def _agg_body(adj_ref, adjf_ref, s_ref, f_ref, wflat_ref, ublk_ref,
              fcwt_ref, fcb_ref, wback_ref,
              emb_ref, emba_ref, h_ref, e_ref, zs_acc, zf_acc):
    k = pl.program_id(1)
    nk = pl.num_programs(1)

    @pl.when(k == 0)
    def _():
        zs_acc[...] = jnp.zeros_like(zs_acc)
        zf_acc[...] = jnp.zeros_like(zf_acc)

    srows = s_ref[pl.ds(k * _BK2, _BK2), :]
    frows = f_ref[pl.ds(k * _BK2, _BK2), :]
    zs_acc[...] += _dot(adj_ref[...].astype(_BF), srows)
    zf_acc[...] += _dot(adjf_ref[...].astype(_BF), frows)

    @pl.when(k == nk - 1)
    def _():
        zs_full = zs_acc[...]
        zf_full = zf_acc[...]
        wflat = wflat_ref[...]
        ublk = ublk_ref[...]
        fcwt = fcwt_ref[...]
        fcb = fcb_ref[...]

        def mha(zsb, zfb):
            s0 = _dot(jnp.tanh(_dot(zsb, wflat)), ublk)
            s1 = _dot(jnp.tanh(_dot(zfb, wflat)), ublk)
            m = jnp.maximum(s0, s1)
            e0 = jnp.exp(s0 - m)
            e1 = jnp.exp(s1 - m)
            den = e0 + e1
            a0 = e0 / den
            a1 = e1 / den
            parts = []
            for h in range(HEADS):
                parts.append(a0[:, h:h + 1] * zsb + a1[:, h:h + 1] * zfb)
            cat = jnp.concatenate(parts, axis=1)
            return _dot(cat, fcwt) + fcb

        emb = mha(zs_full[:, :OUT], zf_full[:, :OUT])
        emba = mha(zs_full[:, OUT:], zf_full[:, OUT:])
        emb_ref[...] = emb
        emba_ref[...] = emba
        h_ref[...] = _dot(emb, wback_ref[...])
        e_ref[...] = jnp.concatenate([emb, emba], axis=1).astype(_BF)


def _read_body(gn_ref, e_ref, emb_ref, emba_ref, dt_ref,
               ret_ref, reta_ref, vs_acc, rs_acc):
    k = pl.program_id(1)
    nk = pl.num_programs(1)

    @pl.when(k == 0)
    def _():
        vs_acc[...] = jnp.zeros_like(vs_acc)
        rs_acc[...] = jnp.zeros_like(rs_acc)

    gnb = gn_ref[...]
    erows = e_ref[pl.ds(k * _BK3, _BK3), :]
    vs_acc[...] += _dot(gnb.astype(_BF), erows)
    rs_acc[...] += jnp.sum(gnb, axis=1, keepdims=True)

    @pl.when(k == nk - 1)
    def _():
        vs = vs_acc[...]
        rs = rs_acc[...]
        emb = emb_ref[...]
        emba = emba_ref[...]
        dt = dt_ref[...]

        def readg(v):
            g = v / rs
            nrm = jnp.maximum(
                jnp.sqrt(jnp.sum(g * g, axis=1, keepdims=True)), 1e-12)
            return jax.nn.sigmoid(g / nrm)

        g1 = readg(vs[:, :OUT])
        g2 = readg(vs[:, OUT:])
        cg = _dot(g1, dt)
        cga = _dot(g2, dt)
        sc1 = jnp.sum(emb * cg, axis=1, keepdims=True)
        sc2 = jnp.sum(emba * cg, axis=1, keepdims=True)
        sa1 = jnp.sum(emba * cga, axis=1, keepdims=True)
        sa2 = jnp.sum(emb * cga, axis=1, keepdims=True)
        ret_ref[...] = jnp.concatenate([sc1, sc2], axis=1)
        reta_ref[...] = jnp.concatenate([sa1, sa2], axis=1)


def kernel(feat, feat_a, adj, adj_feat, graph_neigh, weight_spatial,
           weight_feat, weight_back, spatial_weight, feature_weight,
           w_omega, u_omega, fc_w, fc_b, disc_w, disc_b):
    wsf = jnp.concatenate([weight_spatial * spatial_weight,
                           weight_feat * feature_weight], axis=1)
    wflat = jnp.transpose(w_omega, (1, 0, 2)).reshape(OUT, OUT)
    uflat = u_omega[:, :, 0].reshape(-1)
    ublk = jnp.repeat(jnp.eye(HEADS, dtype=_F32), HD, axis=0) * uflat[:, None]
    ublk = jnp.pad(ublk, ((0, 0), (0, 128 - HEADS)))
    fcwt = fc_w.T
    fcb2 = fc_b.reshape(1, OUT)
    dt = disc_w[0].T

    s, f = pl.pallas_call(
        _proj_body,
        grid=(N // _BM1,),
        in_specs=[
            pl.BlockSpec((_BM1, IN), lambda m: (m, 0)),
            pl.BlockSpec((_BM1, IN), lambda m: (m, 0)),
            pl.BlockSpec((IN, 2 * OUT), lambda m: (0, 0)),
        ],
        out_specs=[
            pl.BlockSpec((_BM1, 2 * OUT), lambda m: (m, 0)),
            pl.BlockSpec((_BM1, 2 * OUT), lambda m: (m, 0)),
        ],
        out_shape=[
            jax.ShapeDtypeStruct((N, 2 * OUT), _BF),
            jax.ShapeDtypeStruct((N, 2 * OUT), _BF),
        ],
        compiler_params=pltpu.CompilerParams(
            dimension_semantics=("parallel",)),
    )(feat, feat_a, wsf)

    grid2 = (N // _BM2, N // _BK2)
    emb, emba, h, e = pl.pallas_call(
        _agg_body,
        grid=grid2,
        in_specs=[
            pl.BlockSpec((_BM2, _BK2), lambda m, k: (m, k)),
            pl.BlockSpec((_BM2, _BK2), lambda m, k: (m, k)),
            pl.BlockSpec((N, 2 * OUT), lambda m, k: (0, 0)),
            pl.BlockSpec((N, 2 * OUT), lambda m, k: (0, 0)),
            pl.BlockSpec((OUT, OUT), lambda m, k: (0, 0)),
            pl.BlockSpec((OUT, 128), lambda m, k: (0, 0)),
            pl.BlockSpec((4 * OUT, OUT), lambda m, k: (0, 0)),
            pl.BlockSpec((1, OUT), lambda m, k: (0, 0)),
            pl.BlockSpec((OUT, IN), lambda m, k: (0, 0)),
        ],
        out_specs=[
            pl.BlockSpec((_BM2, OUT), lambda m, k: (m, 0)),
            pl.BlockSpec((_BM2, OUT), lambda m, k: (m, 0)),
            pl.BlockSpec((_BM2, IN), lambda m, k: (m, 0)),
            pl.BlockSpec((_BM2, 2 * OUT), lambda m, k: (m, 0)),
        ],
        out_shape=[
            jax.ShapeDtypeStruct((N, OUT), _F32),
            jax.ShapeDtypeStruct((N, OUT), _F32),
            jax.ShapeDtypeStruct((N, IN), _F32),
            jax.ShapeDtypeStruct((N, 2 * OUT), _BF),
        ],
        scratch_shapes=[
            pltpu.VMEM((_BM2, 2 * OUT), _F32),
            pltpu.VMEM((_BM2, 2 * OUT), _F32),
        ],
        compiler_params=pltpu.CompilerParams(
            dimension_semantics=("parallel", "arbitrary")),
    )(adj, adj_feat, s, f, wflat, ublk, fcwt, fcb2, weight_back)

    grid3 = (N // _BM3, N // _BK3)
    ret, reta = pl.pallas_call(
        _read_body,
        grid=grid3,
        in_specs=[
            pl.BlockSpec((_BM3, _BK3), lambda m, k: (m, k)),
            pl.BlockSpec((N, 2 * OUT), lambda m, k: (0, 0)),
            pl.BlockSpec((_BM3, OUT), lambda m, k: (m, 0)),
            pl.BlockSpec((_BM3, OUT), lambda m, k: (m, 0)),
            pl.BlockSpec((OUT, OUT), lambda m, k: (0, 0)),
        ],
        out_specs=[
            pl.BlockSpec((_BM3, 2), lambda m, k: (m, 0)),
            pl.BlockSpec((_BM3, 2), lambda m, k: (m, 0)),
        ],
        out_shape=[
            jax.ShapeDtypeStruct((N, 2), _F32),
            jax.ShapeDtypeStruct((N, 2), _F32),
        ],
        scratch_shapes=[
            pltpu.VMEM((_BM3, 2 * OUT), _F32),
            pltpu.VMEM((_BM3, 1), _F32),
        ],
        compiler_params=pltpu.CompilerParams(
            dimension_semantics=("parallel", "arbitrary")),
    )(graph_neigh, e, emb, emba, dt)

    ret = ret + disc_b
    reta = reta + disc_b
    return (emb, h, ret, reta)


# speedup vs baseline: 2.8629x; 1.3023x over previous
import jax
import jax.numpy as jnp
from jax.experimental import pallas as pl
from jax.experimental.pallas import tpu as pltpu

N = 4096
IN = 512
OUT = 256
HEADS = 4
HD = OUT // HEADS

_BF = jnp.bfloat16
_F32 = jnp.float32

_BM1 = 1024
_BM2, _BK2 = 256, 4096
_BM3, _BK3 = 256, 4096


def _dot(a, b):
    return jnp.dot(a, b, preferred_element_type=_F32)


def _proj_body(feat_ref, feata_ref, wsf_ref, s_ref, f_ref):
    w = wsf_ref[...]
    p = _dot(feat_ref[...], w)
    pa = _dot(feata_ref[...], w)
    s_ref[...] = jnp.concatenate([p[:, :OUT], pa[:, :OUT]], axis=1).astype(_BF)
    f_ref[...] = jnp.concatenate([p[:, OUT:], pa[:, OUT:]], axis=1).astype(_BF)


def _agg_body(adj_ref, adjf_ref, s_ref, f_ref, wflat_ref, ublk_ref,
              fcwt_ref, fcb_ref, wback_ref,
              emb_ref, emba_ref, h_ref, e_ref, zs_acc, zf_acc):
    k = pl.program_id(1)
    nk = pl.num_programs(1)

    @pl.when(k == 0)
    def _():
        zs_acc[...] = jnp.zeros_like(zs_acc)
        zf_acc[...] = jnp.zeros_like(zf_acc)

    srows = s_ref[pl.ds(k * _BK2, _BK2), :]
    frows = f_ref[pl.ds(k * _BK2, _BK2), :]
    zs_acc[...] += _dot(adj_ref[...].astype(_BF), srows)
    zf_acc[...] += _dot(adjf_ref[...].astype(_BF), frows)

    @pl.when(k == nk - 1)
    def _():
        zs_full = zs_acc[...]
        zf_full = zf_acc[...]
        wflat = wflat_ref[...]
        ublk = ublk_ref[...]
        fcwt = fcwt_ref[...]
        fcb = fcb_ref[...]

        def mha(zsb, zfb):
            s0 = _dot(jnp.tanh(_dot(zsb, wflat)), ublk)
            s1 = _dot(jnp.tanh(_dot(zfb, wflat)), ublk)
            m = jnp.maximum(s0, s1)
            e0 = jnp.exp(s0 - m)
            e1 = jnp.exp(s1 - m)
            den = e0 + e1
            a0 = e0 / den
            a1 = e1 / den
            parts = []
            for h in range(HEADS):
                parts.append(a0[:, h:h + 1] * zsb + a1[:, h:h + 1] * zfb)
            cat = jnp.concatenate(parts, axis=1)
            return _dot(cat, fcwt) + fcb

        emb = mha(zs_full[:, :OUT], zf_full[:, :OUT])
        emba = mha(zs_full[:, OUT:], zf_full[:, OUT:])
        emb_ref[...] = emb
        emba_ref[...] = emba
        h_ref[...] = _dot(emb, wback_ref[...])
        e_ref[...] = jnp.concatenate([emb, emba], axis=1).astype(_BF)


def _read_body(gn_ref, e_ref, emb_ref, emba_ref, dt_ref,
               ret_ref, reta_ref, vs_acc, rs_acc):
    k = pl.program_id(1)
    nk = pl.num_programs(1)

    @pl.when(k == 0)
    def _():
        vs_acc[...] = jnp.zeros_like(vs_acc)
        rs_acc[...] = jnp.zeros_like(rs_acc)

    gnb = gn_ref[...]
    erows = e_ref[pl.ds(k * _BK3, _BK3), :]
    vs_acc[...] += _dot(gnb.astype(_BF), erows)
    rs_acc[...] += jnp.sum(gnb, axis=1, keepdims=True)

    @pl.when(k == nk - 1)
    def _():
        vs = vs_acc[...]
        rs = rs_acc[...]
        emb = emb_ref[...]
        emba = emba_ref[...]
        dt = dt_ref[...]

        def readg(v):
            g = v / rs
            nrm = jnp.maximum(
                jnp.sqrt(jnp.sum(g * g, axis=1, keepdims=True)), 1e-12)
            return jax.nn.sigmoid(g / nrm)

        g1 = readg(vs[:, :OUT])
        g2 = readg(vs[:, OUT:])
        cg = _dot(g1, dt)
        cga = _dot(g2, dt)
        sc1 = jnp.sum(emb * cg, axis=1, keepdims=True)
        sc2 = jnp.sum(emba * cg, axis=1, keepdims=True)
        sa1 = jnp.sum(emba * cga, axis=1, keepdims=True)
        sa2 = jnp.sum(emb * cga, axis=1, keepdims=True)
        ret_ref[...] = jnp.concatenate([sc1, sc2], axis=1)
        reta_ref[...] = jnp.concatenate([sa1, sa2], axis=1)


def kernel(feat, feat_a, adj, adj_feat, graph_neigh, weight_spatial,
           weight_feat, weight_back, spatial_weight, feature_weight,
           w_omega, u_omega, fc_w, fc_b, disc_w, disc_b):
    wsf = jnp.concatenate([weight_spatial * spatial_weight,
                           weight_feat * feature_weight], axis=1)
    wflat = jnp.transpose(w_omega, (1, 0, 2)).reshape(OUT, OUT)
    uflat = u_omega[:, :, 0].reshape(-1)
    ublk = jnp.repeat(jnp.eye(HEADS, dtype=_F32), HD, axis=0) * uflat[:, None]
    ublk = jnp.pad(ublk, ((0, 0), (0, 128 - HEADS)))
    fcwt = fc_w.T
    fcb2 = fc_b.reshape(1, OUT)
    dt = disc_w[0].T

    s, f = pl.pallas_call(
        _proj_body,
        grid=(N // _BM1,),
        in_specs=[
            pl.BlockSpec((_BM1, IN), lambda m: (m, 0)),
            pl.BlockSpec((_BM1, IN), lambda m: (m, 0)),
            pl.BlockSpec((IN, 2 * OUT), lambda m: (0, 0)),
        ],
        out_specs=[
            pl.BlockSpec((_BM1, 2 * OUT), lambda m: (m, 0)),
            pl.BlockSpec((_BM1, 2 * OUT), lambda m: (m, 0)),
        ],
        out_shape=[
            jax.ShapeDtypeStruct((N, 2 * OUT), _BF),
            jax.ShapeDtypeStruct((N, 2 * OUT), _BF),
        ],
        compiler_params=pltpu.CompilerParams(
            dimension_semantics=("parallel",)),
    )(feat, feat_a, wsf)

    grid2 = (N // _BM2, N // _BK2)
    emb, emba, h, e = pl.pallas_call(
        _agg_body,
        grid=grid2,
        in_specs=[
            pl.BlockSpec((_BM2, _BK2), lambda m, k: (m, k)),
            pl.BlockSpec((_BM2, _BK2), lambda m, k: (m, k)),
            pl.BlockSpec((N, 2 * OUT), lambda m, k: (0, 0)),
            pl.BlockSpec((N, 2 * OUT), lambda m, k: (0, 0)),
            pl.BlockSpec((OUT, OUT), lambda m, k: (0, 0)),
            pl.BlockSpec((OUT, 128), lambda m, k: (0, 0)),
            pl.BlockSpec((4 * OUT, OUT), lambda m, k: (0, 0)),
            pl.BlockSpec((1, OUT), lambda m, k: (0, 0)),
            pl.BlockSpec((OUT, IN), lambda m, k: (0, 0)),
        ],
        out_specs=[
            pl.BlockSpec((_BM2, OUT), lambda m, k: (m, 0)),
            pl.BlockSpec((_BM2, OUT), lambda m, k: (m, 0)),
            pl.BlockSpec((_BM2, IN), lambda m, k: (m, 0)),
            pl.BlockSpec((_BM2, 2 * OUT), lambda m, k: (m, 0)),
        ],
        out_shape=[
            jax.ShapeDtypeStruct((N, OUT), _F32),
            jax.ShapeDtypeStruct((N, OUT), _F32),
            jax.ShapeDtypeStruct((N, IN), _F32),
            jax.ShapeDtypeStruct((N, 2 * OUT), _BF),
        ],
        scratch_shapes=[
            pltpu.VMEM((_BM2, 2 * OUT), _F32),
            pltpu.VMEM((_BM2, 2 * OUT), _F32),
        ],
        compiler_params=pltpu.CompilerParams(
            dimension_semantics=("parallel", "arbitrary")),
    )(adj, adj_feat, s, f, wflat, ublk, fcwt, fcb2, weight_back)

    grid3 = (N // _BM3, N // _BK3)
    ret, reta = pl.pallas_call(
        _read_body,
        grid=grid3,
        in_specs=[
            pl.BlockSpec((_BM3, _BK3), lambda m, k: (m, k)),
            pl.BlockSpec((N, 2 * OUT), lambda m, k: (0, 0)),
            pl.BlockSpec((_BM3, OUT), lambda m, k: (m, 0)),
            pl.BlockSpec((_BM3, OUT), lambda m, k: (m, 0)),
            pl.BlockSpec((OUT, OUT), lambda m, k: (0, 0)),
        ],
        out_specs=[
            pl.BlockSpec((_BM3, 2), lambda m, k: (m, 0)),
            pl.BlockSpec((_BM3, 2), lambda m, k: (m, 0)),
        ],
        out_shape=[
            jax.ShapeDtypeStruct((N, 2), _F32),
            jax.ShapeDtypeStruct((N, 2), _F32),
        ],
        scratch_shapes=[
            pltpu.VMEM((_BM3, 2 * OUT), _F32),
            pltpu.VMEM((_BM3, 1), _F32),
        ],
        compiler_params=pltpu.CompilerParams(
            dimension_semantics=("parallel", "arbitrary")),
    )(graph_neigh, e, emb, emba, dt)

    ret = ret + disc_b
    reta = reta + disc_b
    return (emb, h, ret, reta)


# speedup vs baseline: 3.0061x; 1.0500x over previous
import jax
import jax.numpy as jnp
from jax.experimental import pallas as pl
from jax.experimental.pallas import tpu as pltpu

N = 4096
IN = 512
OUT = 256
HEADS = 4
HD = OUT // HEADS

_BF = jnp.bfloat16
_F32 = jnp.float32

_BM1 = 1024
_BM2, _BK2 = 512, 4096
_BM3, _BK3 = 512, 4096


def _dot(a, b):
    return jnp.dot(a, b, preferred_element_type=_F32)


def _proj_body(feat_ref, feata_ref, wsf_ref, s_ref, f_ref):
    w = wsf_ref[...]
    p = _dot(feat_ref[...], w)
    pa = _dot(feata_ref[...], w)
    s_ref[...] = jnp.concatenate([p[:, :OUT], pa[:, :OUT]], axis=1).astype(_BF)
    f_ref[...] = jnp.concatenate([p[:, OUT:], pa[:, OUT:]], axis=1).astype(_BF)


def _agg_body(adj_ref, adjf_ref, s_ref, f_ref, wflat_ref, ublk_ref,
              fcwt_ref, fcb_ref, wback_ref,
              emb_ref, emba_ref, h_ref, e_ref, zs_acc, zf_acc):
    k = pl.program_id(1)
    nk = pl.num_programs(1)

    @pl.when(k == 0)
    def _():
        zs_acc[...] = jnp.zeros_like(zs_acc)
        zf_acc[...] = jnp.zeros_like(zf_acc)

    srows = s_ref[pl.ds(k * _BK2, _BK2), :]
    frows = f_ref[pl.ds(k * _BK2, _BK2), :]
    zs_acc[...] += _dot(adj_ref[...].astype(_BF), srows)
    zf_acc[...] += _dot(adjf_ref[...].astype(_BF), frows)

    @pl.when(k == nk - 1)
    def _():
        zs_full = zs_acc[...]
        zf_full = zf_acc[...]
        wflat = wflat_ref[...]
        ublk = ublk_ref[...]
        fcwt = fcwt_ref[...]
        fcb = fcb_ref[...]

        def mha(zsb, zfb):
            s0 = _dot(jnp.tanh(_dot(zsb, wflat)), ublk)
            s1 = _dot(jnp.tanh(_dot(zfb, wflat)), ublk)
            m = jnp.maximum(s0, s1)
            e0 = jnp.exp(s0 - m)
            e1 = jnp.exp(s1 - m)
            den = e0 + e1
            a0 = e0 / den
            a1 = e1 / den
            parts = []
            for h in range(HEADS):
                parts.append(a0[:, h:h + 1] * zsb + a1[:, h:h + 1] * zfb)
            cat = jnp.concatenate(parts, axis=1)
            return _dot(cat, fcwt) + fcb

        emb = mha(zs_full[:, :OUT], zf_full[:, :OUT])
        emba = mha(zs_full[:, OUT:], zf_full[:, OUT:])
        emb_ref[...] = emb
        emba_ref[...] = emba
        h_ref[...] = _dot(emb, wback_ref[...])
        e_ref[...] = jnp.concatenate([emb, emba], axis=1).astype(_BF)


def _read_body(gn_ref, e_ref, emb_ref, emba_ref, dt_ref,
               ret_ref, reta_ref, vs_acc, rs_acc):
    k = pl.program_id(1)
    nk = pl.num_programs(1)

    @pl.when(k == 0)
    def _():
        vs_acc[...] = jnp.zeros_like(vs_acc)
        rs_acc[...] = jnp.zeros_like(rs_acc)

    gnb = gn_ref[...]
    erows = e_ref[pl.ds(k * _BK3, _BK3), :]
    vs_acc[...] += _dot(gnb.astype(_BF), erows)
    rs_acc[...] += jnp.sum(gnb, axis=1, keepdims=True)

    @pl.when(k == nk - 1)
    def _():
        vs = vs_acc[...]
        rs = rs_acc[...]
        emb = emb_ref[...]
        emba = emba_ref[...]
        dt = dt_ref[...]

        def readg(v):
            g = v / rs
            nrm = jnp.maximum(
                jnp.sqrt(jnp.sum(g * g, axis=1, keepdims=True)), 1e-12)
            return jax.nn.sigmoid(g / nrm)

        g1 = readg(vs[:, :OUT])
        g2 = readg(vs[:, OUT:])
        cg = _dot(g1, dt)
        cga = _dot(g2, dt)
        sc1 = jnp.sum(emb * cg, axis=1, keepdims=True)
        sc2 = jnp.sum(emba * cg, axis=1, keepdims=True)
        sa1 = jnp.sum(emba * cga, axis=1, keepdims=True)
        sa2 = jnp.sum(emb * cga, axis=1, keepdims=True)
        ret_ref[...] = jnp.concatenate([sc1, sc2], axis=1)
        reta_ref[...] = jnp.concatenate([sa1, sa2], axis=1)


def kernel(feat, feat_a, adj, adj_feat, graph_neigh, weight_spatial,
           weight_feat, weight_back, spatial_weight, feature_weight,
           w_omega, u_omega, fc_w, fc_b, disc_w, disc_b):
    wsf = jnp.concatenate([weight_spatial * spatial_weight,
                           weight_feat * feature_weight], axis=1)
    wflat = jnp.transpose(w_omega, (1, 0, 2)).reshape(OUT, OUT)
    uflat = u_omega[:, :, 0].reshape(-1)
    ublk = jnp.repeat(jnp.eye(HEADS, dtype=_F32), HD, axis=0) * uflat[:, None]
    ublk = jnp.pad(ublk, ((0, 0), (0, 128 - HEADS)))
    fcwt = fc_w.T
    fcb2 = fc_b.reshape(1, OUT)
    dt = disc_w[0].T

    s, f = pl.pallas_call(
        _proj_body,
        grid=(N // _BM1,),
        in_specs=[
            pl.BlockSpec((_BM1, IN), lambda m: (m, 0)),
            pl.BlockSpec((_BM1, IN), lambda m: (m, 0)),
            pl.BlockSpec((IN, 2 * OUT), lambda m: (0, 0)),
        ],
        out_specs=[
            pl.BlockSpec((_BM1, 2 * OUT), lambda m: (m, 0)),
            pl.BlockSpec((_BM1, 2 * OUT), lambda m: (m, 0)),
        ],
        out_shape=[
            jax.ShapeDtypeStruct((N, 2 * OUT), _BF),
            jax.ShapeDtypeStruct((N, 2 * OUT), _BF),
        ],
        compiler_params=pltpu.CompilerParams(
            dimension_semantics=("parallel",)),
    )(feat, feat_a, wsf)

    grid2 = (N // _BM2, N // _BK2)
    emb, emba, h, e = pl.pallas_call(
        _agg_body,
        grid=grid2,
        in_specs=[
            pl.BlockSpec((_BM2, _BK2), lambda m, k: (m, k)),
            pl.BlockSpec((_BM2, _BK2), lambda m, k: (m, k)),
            pl.BlockSpec((N, 2 * OUT), lambda m, k: (0, 0)),
            pl.BlockSpec((N, 2 * OUT), lambda m, k: (0, 0)),
            pl.BlockSpec((OUT, OUT), lambda m, k: (0, 0)),
            pl.BlockSpec((OUT, 128), lambda m, k: (0, 0)),
            pl.BlockSpec((4 * OUT, OUT), lambda m, k: (0, 0)),
            pl.BlockSpec((1, OUT), lambda m, k: (0, 0)),
            pl.BlockSpec((OUT, IN), lambda m, k: (0, 0)),
        ],
        out_specs=[
            pl.BlockSpec((_BM2, OUT), lambda m, k: (m, 0)),
            pl.BlockSpec((_BM2, OUT), lambda m, k: (m, 0)),
            pl.BlockSpec((_BM2, IN), lambda m, k: (m, 0)),
            pl.BlockSpec((_BM2, 2 * OUT), lambda m, k: (m, 0)),
        ],
        out_shape=[
            jax.ShapeDtypeStruct((N, OUT), _F32),
            jax.ShapeDtypeStruct((N, OUT), _F32),
            jax.ShapeDtypeStruct((N, IN), _F32),
            jax.ShapeDtypeStruct((N, 2 * OUT), _BF),
        ],
        scratch_shapes=[
            pltpu.VMEM((_BM2, 2 * OUT), _F32),
            pltpu.VMEM((_BM2, 2 * OUT), _F32),
        ],
        compiler_params=pltpu.CompilerParams(
            dimension_semantics=("parallel", "arbitrary")),
    )(adj, adj_feat, s, f, wflat, ublk, fcwt, fcb2, weight_back)

    grid3 = (N // _BM3, N // _BK3)
    ret, reta = pl.pallas_call(
        _read_body,
        grid=grid3,
        in_specs=[
            pl.BlockSpec((_BM3, _BK3), lambda m, k: (m, k)),
            pl.BlockSpec((N, 2 * OUT), lambda m, k: (0, 0)),
            pl.BlockSpec((_BM3, OUT), lambda m, k: (m, 0)),
            pl.BlockSpec((_BM3, OUT), lambda m, k: (m, 0)),
            pl.BlockSpec((OUT, OUT), lambda m, k: (0, 0)),
        ],
        out_specs=[
            pl.BlockSpec((_BM3, 2), lambda m, k: (m, 0)),
            pl.BlockSpec((_BM3, 2), lambda m, k: (m, 0)),
        ],
        out_shape=[
            jax.ShapeDtypeStruct((N, 2), _F32),
            jax.ShapeDtypeStruct((N, 2), _F32),
        ],
        scratch_shapes=[
            pltpu.VMEM((_BM3, 2 * OUT), _F32),
            pltpu.VMEM((_BM3, 1), _F32),
        ],
        compiler_params=pltpu.CompilerParams(
            dimension_semantics=("parallel", "arbitrary")),
    )(graph_neigh, e, emb, emba, dt)

    ret = ret + disc_b
    reta = reta + disc_b
    return (emb, h, ret, reta)
